# Initial kernel scaffold; baseline (speedup 1.0000x reference)
import jax
import jax.numpy as jnp
from jax.experimental import pallas as pl


def kernel(input_sequence, position_ids, item_table, pos_table, ln_weight, ln_bias):
    raise NotImplementedError("write your pallas kernel here")


# speedup vs baseline: 1.4006x; 1.4006x over previous
import functools
import math

import jax
import jax.numpy as jnp
from jax import lax
from jax.experimental import pallas as pl
from jax.experimental.pallas import tpu as pltpu
from jax.experimental.pallas import tpu_sc as plsc

D = 64
CHUNK = 128
NC = 2
NS = 16
NW = NC * NS
EPS = 1e-5 / 64.0


def _xlane_sum(v):
    lanes = lax.iota(jnp.int32, 16)
    dn = lax.GatherDimensionNumbers(
        offset_dims=(), collapsed_slice_dims=(0,), start_index_map=(0,))
    for shift in (8, 4, 2, 1):
        idx = (lanes + shift) & 15
        rot = lax.gather(v, idx[:, None], dn, (1,),
                         mode=lax.GatherScatterMode.PROMISE_IN_BOUNDS)
        v = v + rot
    return v


def _rsqrt(x):
    i = lax.bitcast_convert_type(x, jnp.int32)
    y = lax.bitcast_convert_type(jnp.int32(0x5F3759DF) - (i >> 1), jnp.float32)
    for _ in range(3):
        y = y * (1.5 - 0.5 * x * y * y)
    return y


def _embed_ln(idx_flat, pid_flat, item_table, pos_table, ln_weight, ln_bias):
    b = idx_flat.shape[0]
    per_w = b // NW
    n_chunks = per_w // CHUNK
    mesh = plsc.VectorSubcoreMesh(core_axis_name="c", subcore_axis_name="s")

    @functools.partial(
        pl.kernel,
        out_type=jax.ShapeDtypeStruct((b, D), jnp.float32),
        mesh=mesh,
        compiler_params=pltpu.CompilerParams(use_tc_tiling_on_sc=False),
        scratch_types=[
            pltpu.VMEM((CHUNK,), jnp.int32),
            pltpu.VMEM((CHUNK,), jnp.int32),
            pltpu.VMEM((CHUNK, D), jnp.float32),
            pltpu.VMEM((CHUNK, D), jnp.float32),
            pltpu.VMEM((D,), jnp.float32),
            pltpu.VMEM((D,), jnp.float32),
            pltpu.SemaphoreType.DMA,
            pltpu.SemaphoreType.DMA,
        ],
    )
    def k(idx_hbm, pid_hbm, item_hbm, pos_hbm, w_hbm, b_hbm, out_hbm,
          idx_v, pid_v, item_v, pos_v, w_v, b_v, sem_a, sem_b):
        wid = lax.axis_index("s") * NC + lax.axis_index("c")
        base_w = wid * per_w
        pltpu.sync_copy(w_hbm, w_v)
        pltpu.sync_copy(b_hbm, b_v)
        w_regs = [w_v[pl.ds(16 * j, 16)] for j in range(4)]
        b_regs = [b_v[pl.ds(16 * j, 16)] for j in range(4)]

        def chunk_body(g, carry):
            base = base_w + g * CHUNK
            pltpu.sync_copy(idx_hbm.at[pl.ds(base, CHUNK)], idx_v)
            pltpu.sync_copy(pid_hbm.at[pl.ds(base, CHUNK)], pid_v)
            cp_item = pltpu.async_copy(item_hbm.at[idx_v], item_v, sem_a)
            cp_pos = pltpu.async_copy(pos_hbm.at[pid_v], pos_v, sem_b)
            cp_item.wait()
            cp_pos.wait()

            def row_body(r, rcarry):
                xs = []
                for j in range(4):
                    it = item_v[r, pl.ds(16 * j, 16)]
                    po = pos_v[r, pl.ds(16 * j, 16)]
                    xs.append(it + po * 0.125)
                s = (xs[0] + xs[1]) + (xs[2] + xs[3])
                sq = (xs[0] * xs[0] + xs[1] * xs[1]) + (
                    xs[2] * xs[2] + xs[3] * xs[3])
                mean = _xlane_sum(s) * (1.0 / D)
                var = _xlane_sum(sq) * (1.0 / D) - mean * mean
                a = _rsqrt(jnp.maximum(var, 0.0) + EPS)
                for j in range(4):
                    item_v[r, pl.ds(16 * j, 16)] = (
                        (xs[j] - mean) * a * w_regs[j] + b_regs[j])
                return rcarry

            lax.fori_loop(0, CHUNK, row_body, 0)
            pltpu.sync_copy(item_v, out_hbm.at[pl.ds(base, CHUNK)])
            return carry

        lax.fori_loop(0, n_chunks, chunk_body, 0)

    return k(idx_flat, pid_flat, item_table, pos_table, ln_weight, ln_bias)


def kernel(input_sequence, position_ids, item_table, pos_table, ln_weight, ln_bias):
    seq, batch = input_sequence.shape
    out = _embed_ln(
        input_sequence.reshape(-1),
        position_ids.reshape(-1),
        item_table,
        pos_table,
        ln_weight,
        ln_bias,
    )
    return out.reshape(seq, batch, D)
